# baseline (speedup 1.0000x reference)
import functools

import jax
import jax.numpy as jnp
from jax import lax
from jax.experimental import pallas as pl
from jax.experimental.pallas import tpu as pltpu
from jax.experimental.pallas import tpu_sc as plsc

B = 16384
K = 64
NC = 2
NS = 16
NW = NC * NS
BPW = B // NW
HALF = B // 2


def _issue_row_dmas(t_hbm, idx_v, buf_v, sem):
    iota16 = lax.broadcasted_iota(jnp.int32, (16,), 0)

    def grp(g, carry):
        r_vec = idx_v[pl.ds(g * 16, 16)]
        for j in range(16):
            r = lax.reduce_max(
                jnp.where(iota16 == j, r_vec, 0), (0,))
            pltpu.async_copy(
                t_hbm.at[pl.ds(r, 1), pl.ds(0, K)],
                buf_v.at[pl.ds(g * 16 + j, 1)],
                sem,
            )
        return carry

    lax.fori_loop(0, BPW // 16, grp, 0)


def _sc_gather():
    mesh = plsc.VectorSubcoreMesh(core_axis_name="c", subcore_axis_name="s")

    @functools.partial(
        pl.kernel,
        mesh=mesh,
        out_type=[
            jax.ShapeDtypeStruct((HALF, 2 * K), jnp.float32),
            jax.ShapeDtypeStruct((HALF, 2 * K), jnp.float32),
        ],
        scratch_types=[
            pltpu.VMEM((BPW,), jnp.int32),
            pltpu.VMEM((BPW,), jnp.int32),
            pltpu.VMEM((BPW, K), jnp.float32),
            pltpu.VMEM((BPW, K), jnp.float32),
            pltpu.SemaphoreType.DMA,
            pltpu.SemaphoreType.DMA,
        ],
        compiler_params=pltpu.CompilerParams(
            use_tc_tiling_on_sc=False, needs_layout_passes=False),
    )
    def gather(uidx_hbm, iidx_hbm, ut_hbm, it_hbm, uout_hbm, iout_hbm,
               uidx_v, iidx_v, ubuf_v, ibuf_v, usem, isem):
        wid = lax.axis_index("s") * NC + lax.axis_index("c")
        base = wid * BPW
        pltpu.sync_copy(uidx_hbm.at[pl.ds(base, BPW)], uidx_v)
        pltpu.sync_copy(iidx_hbm.at[pl.ds(base, BPW)], iidx_v)
        _issue_row_dmas(ut_hbm, uidx_v, ubuf_v, usem)
        _issue_row_dmas(it_hbm, iidx_v, ibuf_v, isem)
        pltpu.make_async_copy(
            ut_hbm.at[pl.ds(0, BPW), pl.ds(0, K)], ubuf_v, usem).wait()
        pltpu.make_async_copy(
            it_hbm.at[pl.ds(0, BPW), pl.ds(0, K)], ibuf_v, isem).wait()
        row0 = jnp.bitwise_and(wid, 15) * BPW
        col0 = jnp.right_shift(wid, 4) * K
        pltpu.sync_copy(
            ubuf_v, uout_hbm.at[pl.ds(row0, BPW), pl.ds(col0, K)])
        pltpu.sync_copy(
            ibuf_v, iout_hbm.at[pl.ds(row0, BPW), pl.ds(col0, K)])

    return gather


def _tc_body(ua_ref, ia_ref, u_ref, i_ref, out_ref):
    m = lax.dot_general(ua_ref[...], ia_ref[...],
                        (((1,), (1,)), ((), ())),
                        preferred_element_type=jnp.float32)
    for half in range(2):
        u = u_ref[:, half * K:(half + 1) * K]
        i = i_ref[:, half * K:(half + 1) * K]
        um = jnp.dot(u, m, preferred_element_type=jnp.float32)
        out_ref[half, :] = jnp.sum(um * i, axis=1)


def kernel(user_indices, item_indices, all_user_T, all_user_A, all_item_T, all_item_A):
    uidx = user_indices.astype(jnp.int32)
    iidx = item_indices.astype(jnp.int32)
    u_pack, i_pack = _sc_gather()(uidx, iidx, all_user_T, all_item_T)

    ua = all_user_A[:K]
    ia = all_item_A[:K]
    blk = 1024
    grid = HALF // blk
    halves = pl.pallas_call(
        _tc_body,
        grid=(grid,),
        in_specs=[
            pl.BlockSpec((K, K), lambda g: (0, 0)),
            pl.BlockSpec((K, K), lambda g: (0, 0)),
            pl.BlockSpec((blk, 2 * K), lambda g: (g, 0)),
            pl.BlockSpec((blk, 2 * K), lambda g: (g, 0)),
        ],
        out_specs=pl.BlockSpec((2, blk), lambda g: (0, g)),
        out_shape=jax.ShapeDtypeStruct((2, HALF), jnp.float32),
    )(ua, ia, u_pack, i_pack)
    return halves.reshape(B)
